# Initial kernel scaffold; baseline (speedup 1.0000x reference)
import jax
import jax.numpy as jnp
from jax.experimental import pallas as pl


def kernel(x, edge_index, W1, b1, W2, b2, W3, b3):
    raise NotImplementedError("write your pallas kernel here")


# speedup vs baseline: 6.1497x; 6.1497x over previous
import functools

import jax
import jax.numpy as jnp
from jax import lax
from jax.experimental import pallas as pl
from jax.experimental.pallas import tpu as pltpu
from jax.experimental.pallas import tpu_sc as plsc

N_NODES = 10000
N_HEDGES = 10000
E = 320000
IN_CH = 128
HID = 256
NUM_CLASSES = 64

NC = 2
NS = 16
NW = NC * NS

CHUNK = 128
EPT = -(-E // (NW * CHUNK)) * CHUNK
NCH = EPT // CHUNK
EP = EPT * NW
R = 10240
PAD_ROW = N_NODES
RPT = R // NS


def _make_stage(count_deg: bool):
  mesh = plsc.VectorSubcoreMesh(core_axis_name="c", subcore_axis_name="s")
  out_type = jax.ShapeDtypeStruct((NC, R, 128), jnp.float32)
  if count_deg:
    out_type = (out_type,) + (jax.ShapeDtypeStruct((NC, R), jnp.float32),) * 2
  scratch = [
      pltpu.VMEM((NCH, CHUNK), jnp.int32),
      pltpu.VMEM((NCH, CHUNK), jnp.int32),
      pltpu.VMEM((CHUNK, 128), jnp.float32),
      pltpu.VMEM_SHARED((R, 128), jnp.float32),
      pltpu.SemaphoreType.DMA,
  ]
  if count_deg:
    scratch += [pltpu.VMEM((CHUNK,), jnp.float32),
                pltpu.VMEM_SHARED((R,), jnp.float32),
                pltpu.VMEM_SHARED((R,), jnp.float32)]

  def body(gidx_hbm, sidx_hbm, table_hbm, zeros_hbm, *rest):
    if count_deg:
      (ones_hbm, zeros1_hbm, out_hbm, dp_hbm, bp_hbm,
       gidx_v, sidx_v, rows_v, acc, sem, ones_v, dacc, bacc) = rest
    else:
      out_hbm, gidx_v, sidx_v, rows_v, acc, sem = rest
    c = lax.axis_index("c")
    s = lax.axis_index("s")
    wid = s * NC + c
    row0 = s * RPT

    pltpu.sync_copy(gidx_hbm.at[wid], gidx_v)
    pltpu.sync_copy(sidx_hbm.at[wid], sidx_v)
    pltpu.sync_copy(zeros_hbm.at[pl.ds(row0, RPT)], acc.at[pl.ds(row0, RPT)])

    if count_deg:
      pltpu.sync_copy(ones_hbm, ones_v)
      pltpu.sync_copy(zeros1_hbm.at[pl.ds(row0, RPT)], dacc.at[pl.ds(row0, RPT)])
      pltpu.sync_copy(zeros1_hbm.at[pl.ds(row0, RPT)], bacc.at[pl.ds(row0, RPT)])

    plsc.subcore_barrier()

    def eb(j, _):
      pltpu.async_copy(table_hbm.at[gidx_v.at[j]], rows_v, sem).wait()
      pltpu.sync_copy(rows_v, acc.at[sidx_v.at[j]], add=True)
      if count_deg:
        pltpu.sync_copy(ones_v, dacc.at[gidx_v.at[j]], add=True)
        pltpu.sync_copy(ones_v, bacc.at[sidx_v.at[j]], add=True)
      return 0
    lax.fori_loop(0, NCH, eb, 0)

    plsc.subcore_barrier()
    pltpu.sync_copy(acc.at[pl.ds(row0, RPT)], out_hbm.at[c, pl.ds(row0, RPT)])
    if count_deg:
      pltpu.sync_copy(dacc.at[pl.ds(row0, RPT)], dp_hbm.at[c, pl.ds(row0, RPT)])
      pltpu.sync_copy(bacc.at[pl.ds(row0, RPT)], bp_hbm.at[c, pl.ds(row0, RPT)])

  return pl.kernel(body, out_type=out_type, mesh=mesh, scratch_types=scratch)


_stage_deg = _make_stage(True)
_stage = _make_stage(False)

BLK = 1024


def _recip_body(bp_ref, dp_ref, binv_ref, dinv_ref):
  b = jnp.sum(bp_ref[...], axis=0)
  d = jnp.sum(dp_ref[...], axis=0)
  binv_ref[...] = jnp.where(b > 0, 1.0 / b, 0.0)[:, None]
  dinv_ref[...] = jnp.where(d > 0, 1.0 / d, 0.0)[:, None]


def _recip(bp, dp):
  return pl.pallas_call(
      _recip_body,
      grid=(R // BLK,),
      in_specs=[pl.BlockSpec((NC, BLK), lambda i: (0, i)),
                pl.BlockSpec((NC, BLK), lambda i: (0, i))],
      out_specs=[pl.BlockSpec((BLK, 1), lambda i: (i, 0)),
                 pl.BlockSpec((BLK, 1), lambda i: (i, 0))],
      out_shape=[jax.ShapeDtypeStruct((R, 1), jnp.float32)] * 2,
  )(bp, dp)


def _combine_body(p_ref, sc_ref, t_ref):
  t_ref[...] = (p_ref[0] + p_ref[1]) * sc_ref[...]


def _combine(p, scale):
  return pl.pallas_call(
      _combine_body,
      grid=(R // BLK,),
      in_specs=[pl.BlockSpec((NC, BLK, 128), lambda i: (0, i, 0)),
                pl.BlockSpec((BLK, 1), lambda i: (i, 0))],
      out_specs=pl.BlockSpec((BLK, 128), lambda i: (i, 0)),
      out_shape=jax.ShapeDtypeStruct((R, 128), jnp.float32),
  )(p, scale)


def _mm_relu_body(q_ref, dinv_ref, w_ref, b_ref, h_ref):
  u = (q_ref[0] + q_ref[1]) * dinv_ref[...]
  h = jnp.maximum(jnp.dot(u, w_ref[...],
                          preferred_element_type=jnp.float32) + b_ref[...], 0.0)
  h_ref[0] = h[:, :128]
  h_ref[1] = h[:, 128:]


def _mm_relu(q, dinv, w, b):
  return pl.pallas_call(
      _mm_relu_body,
      grid=(R // BLK,),
      in_specs=[pl.BlockSpec((NC, BLK, 128), lambda i: (0, i, 0)),
                pl.BlockSpec((BLK, 1), lambda i: (i, 0)),
                pl.BlockSpec((IN_CH, HID), lambda i: (0, 0)),
                pl.BlockSpec((1, HID), lambda i: (0, 0))],
      out_specs=pl.BlockSpec((NC, BLK, 128), lambda i: (0, i, 0)),
      out_shape=jax.ShapeDtypeStruct((NC, R, 128), jnp.float32),
  )(q, dinv, w, b)


def _final_body(qa_ref, qb_ref, dinv_ref, w2_ref, b2_ref, w3_ref, b3_ref, o_ref):
  dinv = dinv_ref[...]
  u = jnp.concatenate([(qa_ref[0] + qa_ref[1]) * dinv,
                       (qb_ref[0] + qb_ref[1]) * dinv], axis=1)
  g = jnp.maximum(jnp.dot(u, w2_ref[...],
                          preferred_element_type=jnp.float32) + b2_ref[...], 0.0)
  o_ref[...] = jnp.dot(g, w3_ref[...],
                       preferred_element_type=jnp.float32) + b3_ref[...]


def _final(qa, qb, dinv, w2, b2, w3, b3):
  return pl.pallas_call(
      _final_body,
      grid=(R // BLK,),
      in_specs=[pl.BlockSpec((NC, BLK, 128), lambda i: (0, i, 0)),
                pl.BlockSpec((NC, BLK, 128), lambda i: (0, i, 0)),
                pl.BlockSpec((BLK, 1), lambda i: (i, 0)),
                pl.BlockSpec((HID, HID), lambda i: (0, 0)),
                pl.BlockSpec((1, HID), lambda i: (0, 0)),
                pl.BlockSpec((HID, NUM_CLASSES), lambda i: (0, 0)),
                pl.BlockSpec((1, NUM_CLASSES), lambda i: (0, 0))],
      out_specs=pl.BlockSpec((BLK, NUM_CLASSES), lambda i: (i, 0)),
      out_shape=jax.ShapeDtypeStruct((R, NUM_CLASSES), jnp.float32),
  )(qa, qb, dinv, w2, b2, w3, b3)


def kernel(x, edge_index, W1, b1, W2, b2, W3, b3):
  ei = edge_index.astype(jnp.int32)
  padv = jnp.full((EP - E,), PAD_ROW, jnp.int32)
  src3 = jnp.concatenate([ei[0], padv]).reshape(NW, NCH, CHUNK)
  he3 = jnp.concatenate([ei[1], padv]).reshape(NW, NCH, CHUNK)
  x_pad = jnp.zeros((R, IN_CH), jnp.float32).at[:N_NODES].set(x)
  zeros = jnp.zeros((R, 128), jnp.float32)
  zeros1 = jnp.zeros((R,), jnp.float32)
  ones_c = jnp.ones((CHUNK,), jnp.float32)
  b1r = b1.reshape(1, HID)
  b2r = b2.reshape(1, HID)
  b3r = b3.reshape(1, NUM_CLASSES)

  pe1, dp, bp = _stage_deg(src3, he3, x_pad, zeros, ones_c, zeros1)
  binv, dinv = _recip(bp, dp)
  t1 = _combine(pe1, binv)
  pn1 = _stage(he3, src3, t1, zeros)
  h = _mm_relu(pn1, dinv, W1, b1r)

  pe2a = _stage(src3, he3, h[0], zeros)
  pe2b = _stage(src3, he3, h[1], zeros)
  t2a = _combine(pe2a, binv)
  t2b = _combine(pe2b, binv)
  pn2a = _stage(he3, src3, t2a, zeros)
  pn2b = _stage(he3, src3, t2b, zeros)
  out = _final(pn2a, pn2b, dinv, W2, b2r, W3, b3r)
  return out[:N_NODES]
